# baseline (speedup 1.0000x reference)
import functools

import jax
import jax.numpy as jnp
from jax import lax
from jax.experimental import pallas as pl
from jax.experimental.pallas import tpu as pltpu
from jax.experimental.pallas import tpu_sc as plsc

N_NODES = 10000
N_EDGES = 320000
D = 128

NC = 2
NS = 16
NW = NC * NS
EPW = N_EDGES // NW
CHUNK = 96
NCHUNK = 105
TOT_CHUNKS = NW * NCHUNK
PAD_E = TOT_CHUNKS * CHUNK - N_EDGES
NBUF = 3
NPK = 5
WB_CHUNK = 80
WB_GROUPS = N_NODES // WB_CHUNK
WB_ITERS = -(-WB_GROUPS // NS)
LANES = 16
DSTEPS = D // LANES


def _sc_body(x_hbm, packed_hbm, w_hbm, scal_hbm, out_hbm,
             pk0, pk1, pk2, pk3, pk4, wv0, wv1, wv2,
             r0, r1, r2, scal_v, acc_sh,
             i0, i1, i2, i3, i4, g0, g1, g2, s0, s1, s2, m0, m1, m2):
    c = lax.axis_index("c")
    s = lax.axis_index("s")
    wid = s * NC + c
    base_cid = wid * NCHUNK

    pks = (pk0, pk1, pk2, pk3, pk4)
    wvs = (wv0, wv1, wv2)
    rows = (r0, r1, r2)
    isems = (i0, i1, i2, i3, i4)
    gsems = (g0, g1, g2)
    ssems = (s0, s1, s2)
    wsems = (m0, m1, m2)

    pltpu.sync_copy(scal_hbm, scal_v)

    zero16 = jnp.zeros((LANES,), jnp.float32)

    def zero_row(i, _):
        for d in range(DSTEPS):
            r2[i, pl.ds(d * LANES, LANES)] = zero16
        return 0

    lax.fori_loop(0, WB_CHUNK, zero_row, 0)

    def zero_acc(t, _):
        g = s + t * NS

        @pl.when(g < WB_GROUPS)
        def _():
            pltpu.sync_copy(r2.at[pl.ds(0, WB_CHUNK)],
                            acc_sh.at[pl.ds(g * WB_CHUNK, WB_CHUNK)])

        return 0

    lax.fori_loop(0, WB_ITERS, zero_acc, 0)

    plsc.subcore_barrier()

    scal_vec = scal_v[...]

    def fire_idx(p, t):
        pltpu.async_copy(packed_hbm.at[base_cid + t], pks[p], isems[p])

    def wait_idx(p, t):
        pltpu.make_async_copy(packed_hbm.at[base_cid + t], pks[p],
                              isems[p]).wait()

    def fire_gather(b, p, t):
        pltpu.async_copy(w_hbm.at[base_cid + t], wvs[b], wsems[b])
        pltpu.async_copy(x_hbm.at[pks[p].at[1]], rows[b], gsems[b])

    def wait_gather(b, p, t):
        pltpu.make_async_copy(w_hbm.at[base_cid + t], wvs[b],
                              wsems[b]).wait()
        pltpu.make_async_copy(x_hbm.at[pks[p].at[1]], rows[b],
                              gsems[b]).wait()

    def fire_scatter(b, p, t):
        pltpu.async_copy(rows[b], acc_sh.at[pks[p].at[0]], ssems[b],
                         add=True)

    def wait_scatter(b, p, t):
        pltpu.make_async_copy(rows[b], acc_sh.at[pks[p].at[0]],
                              ssems[b]).wait()

    def scale(b, t):
        def scale_grp(j, _):
            w16 = wvs[b][0, pl.ds(j * LANES, LANES)] * scal_vec
            base_e = j * LANES
            for lane in range(LANES):
                wsc = lax.broadcast_in_dim(w16[lane], (LANES,), ())
                e = base_e + lane
                for d in range(DSTEPS):
                    sl = pl.ds(d * LANES, LANES)
                    rows[b][e, sl] = rows[b][e, sl] * wsc
            return 0

        lax.fori_loop(0, CHUNK // LANES, scale_grp, 0)

    for tt in range(4):
        fire_idx(tt % NPK, tt)
    wait_idx(0, 0)
    fire_gather(0, 0, 0)
    wait_idx(1, 1)
    fire_gather(1, 1, 1)

    def fifteen(t15, _):
        for off in range(NPK * NBUF):
            t = t15 * (NPK * NBUF) + off
            bb = off % NBUF
            pp = off % NPK

            @pl.when(t < NCHUNK)
            def _():
                wait_gather(bb, pp, t)
                scale(bb, t)
                fire_scatter(bb, pp, t)
                b2 = (bb + 2) % NBUF
                p1 = (pp + 4) % NPK
                p2 = (pp + 2) % NPK

                @pl.when(t + 2 < NCHUNK)
                def _():
                    @pl.when(t >= 1)
                    def _():
                        wait_scatter(b2, p1, t - 1)

                    @pl.when(t + 4 < NCHUNK)
                    def _():
                        fire_idx(p1, t + 4)

                    wait_idx(p2, t + 2)
                    fire_gather(b2, p2, t + 2)

        return 0

    lax.fori_loop(0, -(-NCHUNK // (NPK * NBUF)), fifteen, 0)

    for tt in range(NCHUNK - NBUF, NCHUNK):
        wait_scatter(tt % NBUF, tt % NPK, tt)

    plsc.subcore_barrier()

    def writeback(t, _):
        g = s + t * NS

        @pl.when(g < WB_GROUPS)
        def _():
            off = g * WB_CHUNK
            pltpu.sync_copy(acc_sh.at[pl.ds(off, WB_CHUNK)],
                            out_hbm.at[c, pl.ds(off, WB_CHUNK)])

        return 0

    lax.fori_loop(0, WB_ITERS, writeback, 0)


_sc_kernel = functools.partial(
    pl.kernel,
    out_type=jax.ShapeDtypeStruct((NC, N_NODES, D), jnp.float32),
    mesh=plsc.VectorSubcoreMesh(core_axis_name="c", subcore_axis_name="s"),
    scratch_types=[
        pltpu.VMEM((2, CHUNK), jnp.int32),
        pltpu.VMEM((2, CHUNK), jnp.int32),
        pltpu.VMEM((2, CHUNK), jnp.int32),
        pltpu.VMEM((2, CHUNK), jnp.int32),
        pltpu.VMEM((2, CHUNK), jnp.int32),
        pltpu.VMEM((1, CHUNK), jnp.float32),
        pltpu.VMEM((1, CHUNK), jnp.float32),
        pltpu.VMEM((1, CHUNK), jnp.float32),
        pltpu.VMEM((CHUNK, D), jnp.float32),
        pltpu.VMEM((CHUNK, D), jnp.float32),
        pltpu.VMEM((CHUNK, D), jnp.float32),
        pltpu.VMEM((LANES,), jnp.float32),
        pltpu.VMEM_SHARED((N_NODES, D), jnp.float32),
        pltpu.SemaphoreType.DMA,
        pltpu.SemaphoreType.DMA,
        pltpu.SemaphoreType.DMA,
        pltpu.SemaphoreType.DMA,
        pltpu.SemaphoreType.DMA,
        pltpu.SemaphoreType.DMA,
        pltpu.SemaphoreType.DMA,
        pltpu.SemaphoreType.DMA,
        pltpu.SemaphoreType.DMA,
        pltpu.SemaphoreType.DMA,
        pltpu.SemaphoreType.DMA,
        pltpu.SemaphoreType.DMA,
        pltpu.SemaphoreType.DMA,
        pltpu.SemaphoreType.DMA,
    ],
)(_sc_body)


_TC_ROWS = 1000


def _combine_body(p_ref, o_ref):
    a = p_ref[0] + p_ref[1]
    o_ref[...] = jnp.where(a > 0, a, jnp.exp(a) - 1.0)


_combine = pl.pallas_call(
    _combine_body,
    grid=(N_NODES // _TC_ROWS,),
    in_specs=[pl.BlockSpec((NC, _TC_ROWS, D), lambda i: (0, i, 0))],
    out_specs=pl.BlockSpec((_TC_ROWS, D), lambda i: (i, 0)),
    out_shape=jax.ShapeDtypeStruct((N_NODES, D), jnp.float32),
)


def kernel(x, edge_index, edge_weight, scalar):
    zpad = jnp.zeros((PAD_E,), jnp.int32)
    row = jnp.concatenate([edge_index[0].astype(jnp.int32), zpad])
    col = jnp.concatenate([edge_index[1].astype(jnp.int32), zpad])
    packed = jnp.stack(
        [row.reshape(TOT_CHUNKS, CHUNK),
         col.reshape(TOT_CHUNKS, CHUNK)], axis=1)
    w = jnp.concatenate(
        [edge_weight.astype(jnp.float32),
         jnp.zeros((PAD_E,), jnp.float32)]).reshape(TOT_CHUNKS, 1, CHUNK)
    scal16 = jnp.broadcast_to(scalar.astype(jnp.float32), (LANES,))
    partial = _sc_kernel(x, packed, w, scal16)
    return _combine(partial)


# speedup vs baseline: 1.7354x; 1.7354x over previous
import functools

import jax
import jax.numpy as jnp
from jax import lax
from jax.experimental import pallas as pl
from jax.experimental.pallas import tpu as pltpu
from jax.experimental.pallas import tpu_sc as plsc

N_NODES = 10000
N_EDGES = 320000
D = 128

NC = 2
NS = 16
NW = NC * NS
EPW = N_EDGES // NW
CHUNK = 80
NCHUNK = EPW // CHUNK
TOT_CHUNKS = N_EDGES // CHUNK
NBUF = 3
NPK = 5
WB_CHUNK = 80
WB_GROUPS = N_NODES // WB_CHUNK
WB_ITERS = -(-WB_GROUPS // NS)
LANES = 16
DSTEPS = D // LANES


def _sc_body(x_hbm, packed_hbm, w_hbm, scal_hbm, out_hbm,
             pk0, pk1, pk2, pk3, pk4, w_all, r0, r1, r2, scal_v, acc_sh,
             wsem, i0, i1, i2, i3, i4, g0, g1, g2, s0, s1, s2):
    c = lax.axis_index("c")
    s = lax.axis_index("s")
    wid = s * NC + c
    base_cid = wid * NCHUNK

    pks = (pk0, pk1, pk2, pk3, pk4)
    rows = (r0, r1, r2)
    isems = (i0, i1, i2, i3, i4)
    gsems = (g0, g1, g2)
    ssems = (s0, s1, s2)

    pltpu.async_copy(w_hbm.at[pl.ds(base_cid, NCHUNK)], w_all, wsem)
    pltpu.sync_copy(scal_hbm, scal_v)

    zero16 = jnp.zeros((LANES,), jnp.float32)

    def zero_row(i, _):
        for d in range(DSTEPS):
            r2[i, pl.ds(d * LANES, LANES)] = zero16
        return 0

    lax.fori_loop(0, WB_CHUNK, zero_row, 0)

    def zero_acc(t, _):
        g = s + t * NS

        @pl.when(g < WB_GROUPS)
        def _():
            pltpu.sync_copy(r2, acc_sh.at[pl.ds(g * WB_CHUNK, WB_CHUNK)])

        return 0

    lax.fori_loop(0, WB_ITERS, zero_acc, 0)

    pltpu.make_async_copy(w_hbm.at[pl.ds(base_cid, NCHUNK)],
                          w_all, wsem).wait()
    plsc.subcore_barrier()

    scal_vec = scal_v[...]

    def fire_idx(p, t):
        pltpu.async_copy(packed_hbm.at[base_cid + t], pks[p], isems[p])

    def wait_idx(p, t):
        pltpu.make_async_copy(packed_hbm.at[base_cid + t], pks[p],
                              isems[p]).wait()

    def fire_gather(b, p, t):
        pltpu.async_copy(x_hbm.at[pks[p].at[1]], rows[b], gsems[b])

    def wait_gather(b, p, t):
        pltpu.make_async_copy(x_hbm.at[pks[p].at[1]], rows[b],
                              gsems[b]).wait()

    def fire_scatter(b, p, t):
        pltpu.async_copy(rows[b], acc_sh.at[pks[p].at[0]], ssems[b],
                         add=True)

    def wait_scatter(b, p, t):
        pltpu.make_async_copy(rows[b], acc_sh.at[pks[p].at[0]],
                              ssems[b]).wait()

    def scale(b, t):
        def scale_grp(j, _):
            w16 = w_all[t, 0, pl.ds(j * LANES, LANES)] * scal_vec
            base_e = j * LANES
            for lane in range(LANES):
                wsc = lax.broadcast_in_dim(w16[lane], (LANES,), ())
                e = base_e + lane
                for d in range(DSTEPS):
                    sl = pl.ds(d * LANES, LANES)
                    rows[b][e, sl] = rows[b][e, sl] * wsc
            return 0

        lax.fori_loop(0, CHUNK // LANES, scale_grp, 0)

    for tt in range(4):
        fire_idx(tt % NPK, tt)
    wait_idx(0, 0)
    fire_gather(0, 0, 0)
    wait_idx(1, 1)
    fire_gather(1, 1, 1)

    def fifteen(t15, _):
        for off in range(NPK * NBUF):
            t = t15 * (NPK * NBUF) + off
            bb = off % NBUF
            pp = off % NPK

            @pl.when(t < NCHUNK)
            def _():
                wait_gather(bb, pp, t)
                scale(bb, t)
                fire_scatter(bb, pp, t)
                b2 = (bb + 2) % NBUF
                p1 = (pp + 4) % NPK
                p2 = (pp + 2) % NPK

                @pl.when(t + 2 < NCHUNK)
                def _():
                    @pl.when(t >= 1)
                    def _():
                        wait_scatter(b2, p1, t - 1)

                    @pl.when(t + 4 < NCHUNK)
                    def _():
                        fire_idx(p1, t + 4)

                    wait_idx(p2, t + 2)
                    fire_gather(b2, p2, t + 2)

        return 0

    lax.fori_loop(0, -(-NCHUNK // (NPK * NBUF)), fifteen, 0)

    for tt in range(NCHUNK - NBUF, NCHUNK):
        wait_scatter(tt % NBUF, tt % NPK, tt)

    plsc.subcore_barrier()

    def writeback(t, _):
        g = s + t * NS

        @pl.when(g < WB_GROUPS)
        def _():
            off = g * WB_CHUNK
            pltpu.sync_copy(acc_sh.at[pl.ds(off, WB_CHUNK)],
                            out_hbm.at[c, pl.ds(off, WB_CHUNK)])

        return 0

    lax.fori_loop(0, WB_ITERS, writeback, 0)


_sc_kernel = functools.partial(
    pl.kernel,
    out_type=jax.ShapeDtypeStruct((NC, N_NODES, D), jnp.float32),
    mesh=plsc.VectorSubcoreMesh(core_axis_name="c", subcore_axis_name="s"),
    scratch_types=[
        pltpu.VMEM((2, CHUNK), jnp.int32),
        pltpu.VMEM((2, CHUNK), jnp.int32),
        pltpu.VMEM((2, CHUNK), jnp.int32),
        pltpu.VMEM((2, CHUNK), jnp.int32),
        pltpu.VMEM((2, CHUNK), jnp.int32),
        pltpu.VMEM((NCHUNK, 1, CHUNK), jnp.float32),
        pltpu.VMEM((CHUNK, D), jnp.float32),
        pltpu.VMEM((CHUNK, D), jnp.float32),
        pltpu.VMEM((CHUNK, D), jnp.float32),
        pltpu.VMEM((LANES,), jnp.float32),
        pltpu.VMEM_SHARED((N_NODES, D), jnp.float32),
        pltpu.SemaphoreType.DMA,
        pltpu.SemaphoreType.DMA,
        pltpu.SemaphoreType.DMA,
        pltpu.SemaphoreType.DMA,
        pltpu.SemaphoreType.DMA,
        pltpu.SemaphoreType.DMA,
        pltpu.SemaphoreType.DMA,
        pltpu.SemaphoreType.DMA,
        pltpu.SemaphoreType.DMA,
        pltpu.SemaphoreType.DMA,
        pltpu.SemaphoreType.DMA,
        pltpu.SemaphoreType.DMA,
    ],
)(_sc_body)


_TC_ROWS = 1000


def _combine_body(p_ref, o_ref):
    a = p_ref[0] + p_ref[1]
    o_ref[...] = jnp.where(a > 0, a, jnp.exp(a) - 1.0)


_combine = pl.pallas_call(
    _combine_body,
    grid=(N_NODES // _TC_ROWS,),
    in_specs=[pl.BlockSpec((NC, _TC_ROWS, D), lambda i: (0, i, 0))],
    out_specs=pl.BlockSpec((_TC_ROWS, D), lambda i: (i, 0)),
    out_shape=jax.ShapeDtypeStruct((N_NODES, D), jnp.float32),
)


def kernel(x, edge_index, edge_weight, scalar):
    row = edge_index[0].astype(jnp.int32)
    col = edge_index[1].astype(jnp.int32)
    packed = jnp.stack(
        [row.reshape(TOT_CHUNKS, CHUNK),
         col.reshape(TOT_CHUNKS, CHUNK)], axis=1)
    w = edge_weight.astype(jnp.float32).reshape(TOT_CHUNKS, 1, CHUNK)
    scal16 = jnp.broadcast_to(scalar.astype(jnp.float32), (LANES,))
    partial = _sc_kernel(x, packed, w, scal16)
    return _combine(partial)


# speedup vs baseline: 1.8387x; 1.0595x over previous
import functools

import jax
import jax.numpy as jnp
from jax import lax
from jax.experimental import pallas as pl
from jax.experimental.pallas import tpu as pltpu
from jax.experimental.pallas import tpu_sc as plsc

N_NODES = 10000
N_EDGES = 320000
D = 128

NC = 2
NS = 16
NW = NC * NS
EPW = N_EDGES // NW
CHUNK = 80
NCHUNK = EPW // CHUNK
TOT_CHUNKS = N_EDGES // CHUNK
NBUF = 3
NPK = 5
WB_CHUNK = 80
WB_GROUPS = N_NODES // WB_CHUNK
WB_ITERS = -(-WB_GROUPS // NS)
LANES = 16
DSTEPS = D // LANES


def _sc_body(x_hbm, row_hbm, col_hbm, w_hbm, scal_hbm, out_hbm,
             rs0, rs1, rs2, rs3, rs4, cs0, cs1, cs2, cs3, cs4,
             wv0, wv1, wv2, r0, r1, r2, scal_v, acc_sh,
             i0, i1, i2, i3, i4, g0, g1, g2, s0, s1, s2, m0, m1, m2):
    c = lax.axis_index("c")
    s = lax.axis_index("s")
    wid = s * NC + c
    base_e = wid * EPW

    rss = (rs0, rs1, rs2, rs3, rs4)
    css = (cs0, cs1, cs2, cs3, cs4)
    wvs = (wv0, wv1, wv2)
    rows = (r0, r1, r2)
    isems = (i0, i1, i2, i3, i4)
    gsems = (g0, g1, g2)
    ssems = (s0, s1, s2)
    wsems = (m0, m1, m2)

    pltpu.sync_copy(scal_hbm, scal_v)

    zero16 = jnp.zeros((LANES,), jnp.float32)

    def zero_row(i, _):
        for d in range(DSTEPS):
            r2[i, pl.ds(d * LANES, LANES)] = zero16
        return 0

    lax.fori_loop(0, WB_CHUNK, zero_row, 0)

    def zero_acc(t, _):
        g = s + t * NS

        @pl.when(g < WB_GROUPS)
        def _():
            pltpu.sync_copy(r2, acc_sh.at[pl.ds(g * WB_CHUNK, WB_CHUNK)])

        return 0

    lax.fori_loop(0, WB_ITERS, zero_acc, 0)

    plsc.subcore_barrier()

    scal_vec = scal_v[...]

    def fire_idx(p, t):
        sl = pl.ds(base_e + t * CHUNK, CHUNK)
        pltpu.async_copy(row_hbm.at[sl], rss[p], isems[p])
        pltpu.async_copy(col_hbm.at[sl], css[p], isems[p])

    def wait_idx(p, t):
        sl = pl.ds(base_e + t * CHUNK, CHUNK)
        pltpu.make_async_copy(row_hbm.at[sl], rss[p], isems[p]).wait()
        pltpu.make_async_copy(col_hbm.at[sl], css[p], isems[p]).wait()

    def fire_gather(b, p, t):
        pltpu.async_copy(w_hbm.at[pl.ds(base_e + t * CHUNK, CHUNK)],
                         wvs[b], wsems[b])
        pltpu.async_copy(x_hbm.at[css[p]], rows[b], gsems[b])

    def wait_gather(b, p, t):
        pltpu.make_async_copy(w_hbm.at[pl.ds(base_e + t * CHUNK, CHUNK)],
                              wvs[b], wsems[b]).wait()
        pltpu.make_async_copy(x_hbm.at[css[p]], rows[b],
                              gsems[b]).wait()

    def fire_scatter(b, p, t):
        pltpu.async_copy(rows[b], acc_sh.at[rss[p]], ssems[b],
                         add=True)

    def wait_scatter(b, p, t):
        pltpu.make_async_copy(rows[b], acc_sh.at[rss[p]],
                              ssems[b]).wait()

    def scale(b, t):
        def scale_grp(j, _):
            w16 = wvs[b][pl.ds(j * LANES, LANES)] * scal_vec
            base_e = j * LANES
            for lane in range(LANES):
                wsc = lax.broadcast_in_dim(w16[lane], (LANES,), ())
                e = base_e + lane
                for d in range(DSTEPS):
                    sl = pl.ds(d * LANES, LANES)
                    rows[b][e, sl] = rows[b][e, sl] * wsc
            return 0

        lax.fori_loop(0, CHUNK // LANES, scale_grp, 0)

    for tt in range(4):
        fire_idx(tt % NPK, tt)
    wait_idx(0, 0)
    fire_gather(0, 0, 0)
    wait_idx(1, 1)
    fire_gather(1, 1, 1)

    def fifteen(t15, _):
        for off in range(NPK * NBUF):
            t = t15 * (NPK * NBUF) + off
            bb = off % NBUF
            pp = off % NPK

            @pl.when(t < NCHUNK)
            def _():
                wait_gather(bb, pp, t)
                scale(bb, t)
                fire_scatter(bb, pp, t)
                b2 = (bb + 2) % NBUF
                p1 = (pp + 4) % NPK
                p2 = (pp + 2) % NPK

                @pl.when(t + 2 < NCHUNK)
                def _():
                    @pl.when(t >= 1)
                    def _():
                        wait_scatter(b2, p1, t - 1)

                    @pl.when(t + 4 < NCHUNK)
                    def _():
                        fire_idx(p1, t + 4)

                    wait_idx(p2, t + 2)
                    fire_gather(b2, p2, t + 2)

        return 0

    lax.fori_loop(0, -(-NCHUNK // (NPK * NBUF)), fifteen, 0)

    for tt in range(NCHUNK - NBUF, NCHUNK):
        wait_scatter(tt % NBUF, tt % NPK, tt)

    plsc.subcore_barrier()

    def writeback(t, _):
        g = s + t * NS

        @pl.when(g < WB_GROUPS)
        def _():
            off = g * WB_CHUNK
            pltpu.sync_copy(acc_sh.at[pl.ds(off, WB_CHUNK)],
                            out_hbm.at[c, pl.ds(off, WB_CHUNK)])

        return 0

    lax.fori_loop(0, WB_ITERS, writeback, 0)


_sc_kernel = functools.partial(
    pl.kernel,
    out_type=jax.ShapeDtypeStruct((NC, N_NODES, D), jnp.float32),
    mesh=plsc.VectorSubcoreMesh(core_axis_name="c", subcore_axis_name="s"),
    scratch_types=[
        pltpu.VMEM((CHUNK,), jnp.int32),
        pltpu.VMEM((CHUNK,), jnp.int32),
        pltpu.VMEM((CHUNK,), jnp.int32),
        pltpu.VMEM((CHUNK,), jnp.int32),
        pltpu.VMEM((CHUNK,), jnp.int32),
        pltpu.VMEM((CHUNK,), jnp.int32),
        pltpu.VMEM((CHUNK,), jnp.int32),
        pltpu.VMEM((CHUNK,), jnp.int32),
        pltpu.VMEM((CHUNK,), jnp.int32),
        pltpu.VMEM((CHUNK,), jnp.int32),
        pltpu.VMEM((CHUNK,), jnp.float32),
        pltpu.VMEM((CHUNK,), jnp.float32),
        pltpu.VMEM((CHUNK,), jnp.float32),
        pltpu.VMEM((CHUNK, D), jnp.float32),
        pltpu.VMEM((CHUNK, D), jnp.float32),
        pltpu.VMEM((CHUNK, D), jnp.float32),
        pltpu.VMEM((LANES,), jnp.float32),
        pltpu.VMEM_SHARED((N_NODES, D), jnp.float32),
        pltpu.SemaphoreType.DMA,
        pltpu.SemaphoreType.DMA,
        pltpu.SemaphoreType.DMA,
        pltpu.SemaphoreType.DMA,
        pltpu.SemaphoreType.DMA,
        pltpu.SemaphoreType.DMA,
        pltpu.SemaphoreType.DMA,
        pltpu.SemaphoreType.DMA,
        pltpu.SemaphoreType.DMA,
        pltpu.SemaphoreType.DMA,
        pltpu.SemaphoreType.DMA,
        pltpu.SemaphoreType.DMA,
        pltpu.SemaphoreType.DMA,
        pltpu.SemaphoreType.DMA,
    ],
)(_sc_body)


_TC_ROWS = 1000


def _combine_body(p_ref, o_ref):
    a = p_ref[0] + p_ref[1]
    o_ref[...] = jnp.where(a > 0, a, jnp.exp(a) - 1.0)


_combine = pl.pallas_call(
    _combine_body,
    grid=(N_NODES // _TC_ROWS,),
    in_specs=[pl.BlockSpec((NC, _TC_ROWS, D), lambda i: (0, i, 0))],
    out_specs=pl.BlockSpec((_TC_ROWS, D), lambda i: (i, 0)),
    out_shape=jax.ShapeDtypeStruct((N_NODES, D), jnp.float32),
)


def kernel(x, edge_index, edge_weight, scalar):
    row = edge_index[0].astype(jnp.int32)
    col = edge_index[1].astype(jnp.int32)
    w = edge_weight.astype(jnp.float32)
    scal16 = jnp.broadcast_to(scalar.astype(jnp.float32), (LANES,))
    partial = _sc_kernel(x, row, col, w, scal16)
    return _combine(partial)
